# Initial kernel scaffold; baseline (speedup 1.0000x reference)
import jax
import jax.numpy as jnp
from jax.experimental import pallas as pl


def kernel(x, index, dim_size):
    raise NotImplementedError("write your pallas kernel here")


# speedup vs baseline: 1.4281x; 1.4281x over previous
import jax
import jax.numpy as jnp
from jax import lax
from jax.experimental import pallas as pl
from jax.experimental.pallas import tpu as pltpu
from jax.experimental.pallas import tpu_sc as plsc

N_EDGES = 160000
D_FEAT = 256
N_SEG = 10000
NC = 2
NS = 16
NW = NC * NS
W_MAX = 376
ACC_ROWS = 380
CHUNK = 128
LANES = 16
BOUNDS_LEN = 9 * LANES


def _seg_sum_body(x_hbm, idx_hbm, bounds_hbm, out_hbm, bounds_v, rows_v,
                  idx_v, acc_v):
    c = lax.axis_index("c")
    s = lax.axis_index("s")
    wid = c * NS + s

    pltpu.sync_copy(bounds_hbm, bounds_v)
    elo = bounds_v[pl.ds(wid, LANES)][0]
    ehi = bounds_v[pl.ds(NW + wid, LANES)][0]
    glo = bounds_v[pl.ds(2 * NW + wid, LANES)][0]
    ghi = bounds_v[pl.ds(3 * NW + wid, LANES)][0]
    nrows = ghi - glo

    zeros16 = jnp.zeros((LANES,), jnp.float32)

    def zero_body(i, carry):
        acc_v[pl.ds(i * LANES, LANES)] = zeros16
        return carry

    lax.fori_loop(0, ACC_ROWS * D_FEAT // LANES, zero_body, 0)

    def chunk_body(i, carry):
        start = pl.multiple_of(elo + i * CHUNK, CHUNK)
        pltpu.sync_copy(x_hbm.at[pl.ds(start * D_FEAT, CHUNK * D_FEAT)], rows_v)
        pltpu.sync_copy(idx_hbm.at[pl.ds(start, CHUNK)], idx_v)

        def grp_body(j, carry2):
            jb = pl.multiple_of(j * LANES, LANES)
            v = idx_v[pl.ds(jb, LANES)]
            local = v - glo
            ok = (local >= 0) & (local < nrows)
            wvec = jnp.where(ok, local, W_MAX)
            for l in range(LANES):
                abase = wvec[l] * D_FEAT
                ebase = (jb + l) * D_FEAT
                for g in range(D_FEAT // LANES):
                    val = rows_v[pl.ds(ebase + g * LANES, LANES)]
                    plsc.addupdate(acc_v.at[pl.ds(abase + g * LANES, LANES)], val)
            return carry2

        lax.fori_loop(0, CHUNK // LANES, grp_body, 0)
        return carry

    nchunks = (ehi - elo) // CHUNK
    lax.fori_loop(0, nchunks, chunk_body, 0)

    def out_body(i, carry):
        pltpu.sync_copy(
            acc_v.at[pl.ds(i * LANES * D_FEAT, LANES * D_FEAT)],
            out_hbm.at[pl.ds((glo + i * LANES) * D_FEAT, LANES * D_FEAT)])
        return carry

    n_full = nrows // LANES
    lax.fori_loop(0, n_full, out_body, 0)
    off = n_full * LANES
    for sub in (8, 4, 2, 1):
        take = (nrows // sub) % 2

        @pl.when(take == 1)
        def _copy_rem(off=off, sub=sub):
            o = pl.multiple_of(off * D_FEAT, 8)
            pltpu.sync_copy(
                acc_v.at[pl.ds(o, sub * D_FEAT)],
                out_hbm.at[pl.ds((glo + off) * D_FEAT, sub * D_FEAT)])

        off = off + take * sub


def _seg_sum_sc(x_flat, idx32, bounds):
    mesh = plsc.VectorSubcoreMesh(core_axis_name="c", subcore_axis_name="s")
    k = pl.kernel(
        _seg_sum_body,
        out_type=jax.ShapeDtypeStruct((N_SEG * D_FEAT,), jnp.float32),
        mesh=mesh,
        scratch_types=[
            pltpu.VMEM((BOUNDS_LEN,), jnp.int32),
            pltpu.VMEM((CHUNK * D_FEAT,), jnp.float32),
            pltpu.VMEM((CHUNK,), jnp.int32),
            pltpu.VMEM((ACC_ROWS * D_FEAT,), jnp.float32),
        ],
    )
    return k(x_flat, idx32, bounds)


def kernel(x, index, dim_size):
    del dim_size
    idx32 = index.astype(jnp.int32)

    wq = jnp.arange(1, NW, dtype=jnp.int32) * (N_EDGES // NW)
    q = idx32[wq]

    def scan_body(g_prev, qa):
        qw, w = qa
        lower = jnp.maximum(g_prev, N_SEG - W_MAX * (NW - w))
        g = jnp.clip(qw, lower, g_prev + W_MAX)
        return g, g

    _, g_mid = lax.scan(scan_body, jnp.int32(0),
                        (q, jnp.arange(1, NW, dtype=jnp.int32)))
    g = jnp.concatenate([jnp.zeros((1,), jnp.int32), g_mid,
                         jnp.full((1,), N_SEG, jnp.int32)])

    sedge = jnp.searchsorted(idx32, g).astype(jnp.int32)
    elo = sedge[:NW] // CHUNK * CHUNK
    ehi = (sedge[1:] + CHUNK - 1) // CHUNK * CHUNK
    bounds = jnp.concatenate([
        elo, ehi, g[:NW], g[1:],
        jnp.zeros((LANES,), jnp.int32),
    ])

    out = _seg_sum_sc(x.reshape(-1), idx32, bounds)
    return out.reshape(N_SEG, D_FEAT)


# speedup vs baseline: 1.8032x; 1.2626x over previous
import jax
import jax.numpy as jnp
from jax import lax
from jax.experimental import pallas as pl
from jax.experimental.pallas import tpu as pltpu
from jax.experimental.pallas import tpu_sc as plsc

N_EDGES = 160000
D_FEAT = 256
N_SEG = 10000
NC = 2
NS = 16
NW = NC * NS
W_MAX = 360
ACC_ROWS = 364
CHUNK = 64
LANES = 16
BOUNDS_LEN = 9 * LANES


def _seg_sum_body(x_hbm, idx_hbm, bounds_hbm, out_hbm, bounds_v,
                  rows_v0, rows_v1, idx_v0, idx_v1, acc_v, sem0, sem1):
    c = lax.axis_index("c")
    s = lax.axis_index("s")
    wid = c * NS + s

    pltpu.sync_copy(bounds_hbm, bounds_v)
    elo = bounds_v[pl.ds(wid, LANES)][0]
    ehi = bounds_v[pl.ds(NW + wid, LANES)][0]
    glo = bounds_v[pl.ds(2 * NW + wid, LANES)][0]
    ghi = bounds_v[pl.ds(3 * NW + wid, LANES)][0]
    nrows = ghi - glo
    nchunks = (ehi - elo) // CHUNK

    rows = (rows_v0, rows_v1)
    idxs = (idx_v0, idx_v1)
    sems = (sem0, sem1)

    def start(b, ci):
        st = pl.multiple_of(elo + ci * CHUNK, CHUNK)
        pltpu.async_copy(x_hbm.at[pl.ds(st * D_FEAT, CHUNK * D_FEAT)],
                         rows[b], sems[b])
        pltpu.async_copy(idx_hbm.at[pl.ds(st, CHUNK)], idxs[b], sems[b])

    def wait(b, ci):
        st = pl.multiple_of(elo + ci * CHUNK, CHUNK)
        pltpu.make_async_copy(x_hbm.at[pl.ds(st * D_FEAT, CHUNK * D_FEAT)],
                              rows[b], sems[b]).wait()
        pltpu.make_async_copy(idx_hbm.at[pl.ds(st, CHUNK)],
                              idxs[b], sems[b]).wait()

    @pl.when(nchunks > 0)
    def _prime0():
        start(0, 0)

    @pl.when(nchunks > 1)
    def _prime1():
        start(1, 1)

    zeros16 = jnp.zeros((LANES,), jnp.float32)

    def zero_body(i, carry):
        base = pl.multiple_of(i * LANES * LANES, LANES)
        for u in range(LANES):
            acc_v[pl.ds(base + u * LANES, LANES)] = zeros16
        return carry

    lax.fori_loop(0, ACC_ROWS * D_FEAT // (LANES * LANES), zero_body, 0)

    def compute(b):
        rv, iv = rows[b], idxs[b]

        def grp_body(j, carry2):
            jb = pl.multiple_of(j * LANES, LANES)
            v = iv[pl.ds(jb, LANES)]
            local = v - glo
            ok = (local >= 0) & (local < nrows)
            wvec = jnp.where(ok, local, W_MAX)
            for l in range(LANES):
                abase = wvec[l] * D_FEAT
                ebase = (jb + l) * D_FEAT
                for g in range(D_FEAT // LANES):
                    val = rv[pl.ds(ebase + g * LANES, LANES)]
                    plsc.addupdate(acc_v.at[pl.ds(abase + g * LANES, LANES)],
                                   val)
            return carry2

        lax.fori_loop(0, CHUNK // LANES, grp_body, 0)

    def outer(i2, carry):
        for b in range(2):
            ci = i2 * 2 + b

            @pl.when(ci < nchunks)
            def _step(b=b, ci=ci):
                wait(b, ci)
                compute(b)

                @pl.when(ci + 2 < nchunks)
                def _next():
                    start(b, ci + 2)

        return carry

    lax.fori_loop(0, (nchunks + 1) // 2, outer, 0)

    def out_body(i, carry):
        pltpu.sync_copy(
            acc_v.at[pl.ds(i * LANES * D_FEAT, LANES * D_FEAT)],
            out_hbm.at[pl.ds((glo + i * LANES) * D_FEAT, LANES * D_FEAT)])
        return carry

    n_full = nrows // LANES
    lax.fori_loop(0, n_full, out_body, 0)
    off = n_full * LANES
    for sub in (8, 4, 2, 1):
        take = (nrows // sub) % 2

        @pl.when(take == 1)
        def _copy_rem(off=off, sub=sub):
            o = pl.multiple_of(off * D_FEAT, 8)
            pltpu.sync_copy(
                acc_v.at[pl.ds(o, sub * D_FEAT)],
                out_hbm.at[pl.ds((glo + off) * D_FEAT, sub * D_FEAT)])

        off = off + take * sub


def _seg_sum_sc(x_flat, idx32, bounds):
    mesh = plsc.VectorSubcoreMesh(core_axis_name="c", subcore_axis_name="s")
    k = pl.kernel(
        _seg_sum_body,
        out_type=jax.ShapeDtypeStruct((N_SEG * D_FEAT,), jnp.float32),
        mesh=mesh,
        scratch_types=[
            pltpu.VMEM((BOUNDS_LEN,), jnp.int32),
            pltpu.VMEM((CHUNK * D_FEAT,), jnp.float32),
            pltpu.VMEM((CHUNK * D_FEAT,), jnp.float32),
            pltpu.VMEM((CHUNK,), jnp.int32),
            pltpu.VMEM((CHUNK,), jnp.int32),
            pltpu.VMEM((ACC_ROWS * D_FEAT,), jnp.float32),
            pltpu.SemaphoreType.DMA,
            pltpu.SemaphoreType.DMA,
        ],
    )
    return k(x_flat, idx32, bounds)


def kernel(x, index, dim_size):
    del dim_size
    idx32 = index.astype(jnp.int32)

    wq = jnp.arange(1, NW, dtype=jnp.int32) * (N_EDGES // NW)
    q = idx32[wq]

    def scan_body(g_prev, qa):
        qw, w = qa
        lower = jnp.maximum(g_prev, N_SEG - W_MAX * (NW - w))
        g = jnp.clip(qw, lower, g_prev + W_MAX)
        return g, g

    _, g_mid = lax.scan(scan_body, jnp.int32(0),
                        (q, jnp.arange(1, NW, dtype=jnp.int32)))
    g = jnp.concatenate([jnp.zeros((1,), jnp.int32), g_mid,
                         jnp.full((1,), N_SEG, jnp.int32)])

    sedge = jnp.searchsorted(idx32, g).astype(jnp.int32)
    elo = sedge[:NW] // CHUNK * CHUNK
    ehi = (sedge[1:] + CHUNK - 1) // CHUNK * CHUNK
    bounds = jnp.concatenate([
        elo, ehi, g[:NW], g[1:],
        jnp.zeros((LANES,), jnp.int32),
    ])

    out = _seg_sum_sc(x.reshape(-1), idx32, bounds)
    return out.reshape(N_SEG, D_FEAT)
